# Initial kernel scaffold; baseline (speedup 1.0000x reference)
import jax
import jax.numpy as jnp
from jax.experimental import pallas as pl


def kernel(x, w1, b1, w2, b2):
    raise NotImplementedError("write your pallas kernel here")


# speedup vs baseline: 1.0652x; 1.0652x over previous
import jax
import jax.numpy as jnp
from jax.experimental import pallas as pl
from jax.experimental.pallas import tpu as pltpu


def _pick_row_tile(h, max_rows=16):
    for th in range(min(h, max_rows), 0, -1):
        if h % th == 0:
            return th
    return h


def _shifted(v, o, length):
    m = v.shape[1]
    if o < 0:
        return jnp.concatenate([v[:, :-o], v[:, :length + o]], axis=1)
    if o + length > m:
        return jnp.concatenate([v[:, o:], v[:, :o + length - m]], axis=1)
    return v[:, o:o + length]


def kernel(x, w1, b1, w2, b2):
    n, cin, h, w = x.shape
    cout = w1.shape[0]
    wp = w + 2
    th = _pick_row_tile(h)
    nt = h // th
    r1 = th + 2
    frows = -(-(th + 4) // 8) * 8
    xcols = frows * wp

    xp = jnp.pad(x, ((0, 0), (0, 0), (2, frows - th - 2), (1, 1)))
    xp = xp.astype(jnp.bfloat16)

    w1p = jnp.transpose(w1, (0, 2, 3, 1)).reshape(cout, 9 * cin)
    w1p = w1p.astype(jnp.bfloat16)
    w2p = jnp.transpose(w2, (0, 2, 3, 1)).reshape(cout, 9 * cout)
    w2p = w2p.astype(jnp.bfloat16)
    b1p = jnp.broadcast_to(b1.astype(jnp.float32)[:, None], (cout, 128))
    b2p = jnp.broadcast_to(b2.astype(jnp.float32)[:, None], (cout, 128))

    lane = jnp.arange(r1 * wp, dtype=jnp.int32)[None, None, :]
    col = lane % wp
    good = (col != 0) & (col != wp - 1)
    tt = jnp.arange(nt, dtype=jnp.int32)[:, None, None]
    good = good & ~((tt == 0) & (lane < wp))
    good = good & ~((tt == nt - 1) & (lane >= (r1 - 1) * wp))
    maskp = good.astype(jnp.bfloat16)

    def body(x_hbm, w1_ref, w2_ref, b1_ref, b2_ref, m_ref, o_ref, xbuf):
        b = pl.program_id(0)
        t = pl.program_id(1)

        pltpu.sync_copy(x_hbm.at[b, :, pl.ds(t * th, frows), :], xbuf)
        xv = xbuf[...].reshape(cin, xcols)

        patch1 = jnp.concatenate(
            [_shifted(xv, dy * wp + dx - 1, r1 * wp)
             for dy in range(3) for dx in range(3)], axis=0)
        acc1 = jnp.dot(w1_ref[...], patch1,
                       preferred_element_type=jnp.float32)

        mid = jnp.maximum(acc1 + b1_ref[:, 0:1], 0.0).astype(jnp.bfloat16)
        mid = mid * m_ref[0]

        patch2 = jnp.concatenate(
            [_shifted(mid, dy * wp + dx - 1, th * wp)
             for dy in range(3) for dx in range(3)], axis=0)
        acc2 = jnp.dot(w2_ref[...], patch2,
                       preferred_element_type=jnp.float32)
        y2 = jnp.maximum(acc2 + b2_ref[:, 0:1], 0.0)

        for y in range(th):
            o_ref[0, :, pl.ds(y * w, w)] = y2[:, y * wp + 1:y * wp + 1 + w]

    out = pl.pallas_call(
        body,
        out_shape=jax.ShapeDtypeStruct((n, cout, h * w), x.dtype),
        grid_spec=pltpu.PrefetchScalarGridSpec(
            num_scalar_prefetch=0,
            grid=(n, nt),
            in_specs=[
                pl.BlockSpec(memory_space=pl.ANY),
                pl.BlockSpec((cout, 9 * cin), lambda b, t: (0, 0)),
                pl.BlockSpec((cout, 9 * cout), lambda b, t: (0, 0)),
                pl.BlockSpec((cout, 128), lambda b, t: (0, 0)),
                pl.BlockSpec((cout, 128), lambda b, t: (0, 0)),
                pl.BlockSpec((1, 1, r1 * wp), lambda b, t: (t, 0, 0)),
            ],
            out_specs=pl.BlockSpec((1, cout, th * w), lambda b, t: (b, 0, t)),
            scratch_shapes=[
                pltpu.VMEM((cin, frows, wp), jnp.bfloat16),
            ]),
        compiler_params=pltpu.CompilerParams(
            dimension_semantics=("parallel", "parallel"),
            vmem_limit_bytes=100 * 1024 * 1024),
    )(xp, w1p, w2p, b1p, b2p, maskp)

    return out.reshape(n, cout, h, w)


# speedup vs baseline: 1.3211x; 1.2402x over previous
import jax
import jax.numpy as jnp
from jax.experimental import pallas as pl
from jax.experimental.pallas import tpu as pltpu


def _pick_row_tile(h, max_rows=16):
    for th in range(min(h, max_rows), 0, -1):
        if h % th == 0:
            return th
    return h


def _shifted(v, o, length):
    m = v.shape[1]
    if o < 0:
        return jnp.concatenate([v[:, :-o], v[:, :length + o]], axis=1)
    if o + length > m:
        return jnp.concatenate([v[:, o:], v[:, :o + length - m]], axis=1)
    return v[:, o:o + length]


def kernel(x, w1, b1, w2, b2):
    n, cin, h, w = x.shape
    cout = w1.shape[0]
    wp = w + 2
    th = _pick_row_tile(h)
    nt = h // th
    r1 = th + 2
    frows = -(-(th + 4) // 8) * 8
    xcols = frows * wp

    xp = jnp.pad(x, ((0, 0), (0, 0), (2, frows - th - 2), (1, 1)))
    xp = xp.astype(jnp.bfloat16)

    w1p = jnp.transpose(w1, (0, 2, 3, 1)).reshape(cout, 9 * cin)
    w1p = w1p.astype(jnp.bfloat16)
    w2p = jnp.transpose(w2, (0, 2, 3, 1)).reshape(cout, 9 * cout)
    w2p = w2p.astype(jnp.bfloat16)
    b1p = jnp.broadcast_to(b1.astype(jnp.float32)[:, None], (cout, 128))
    b2p = jnp.broadcast_to(b2.astype(jnp.float32)[:, None], (cout, 128))

    lane = jnp.arange(r1 * wp, dtype=jnp.int32)[None, None, :]
    col = lane % wp
    good = (col != 0) & (col != wp - 1)
    tt = jnp.arange(nt, dtype=jnp.int32)[:, None, None]
    good = good & ~((tt == 0) & (lane < wp))
    good = good & ~((tt == nt - 1) & (lane >= (r1 - 1) * wp))
    maskp = good.astype(jnp.bfloat16)

    def body(x_hbm, w1_ref, w2_ref, b1_ref, b2_ref, m_ref, o_ref, xbuf,
             in_sem):
        b = pl.program_id(0)
        t = pl.program_id(1)
        ntt = pl.num_programs(1)

        def fetch(tt, slot):
            pltpu.make_async_copy(x_hbm.at[b, :, pl.ds(tt * th, frows), :],
                                  xbuf.at[slot], in_sem.at[slot]).start()

        slot = jax.lax.rem(t, 2)

        @pl.when(t == 0)
        def _():
            fetch(t, slot)

        @pl.when(t + 1 < ntt)
        def _():
            fetch(t + 1, 1 - slot)

        pltpu.make_async_copy(x_hbm.at[b, :, pl.ds(0, frows), :],
                              xbuf.at[slot], in_sem.at[slot]).wait()
        xv = xbuf[slot].reshape(cin, xcols)

        patch1 = jnp.concatenate(
            [_shifted(xv, dy * wp + dx - 1, r1 * wp)
             for dy in range(3) for dx in range(3)], axis=0)
        acc1 = jnp.dot(w1_ref[...], patch1,
                       preferred_element_type=jnp.float32)

        mid = jnp.maximum(acc1 + b1_ref[:, 0:1], 0.0).astype(jnp.bfloat16)
        mid = mid * m_ref[0]

        patch2 = jnp.concatenate(
            [_shifted(mid, dy * wp + dx - 1, th * wp)
             for dy in range(3) for dx in range(3)], axis=0)
        acc2 = jnp.dot(w2_ref[...], patch2,
                       preferred_element_type=jnp.float32)
        y2 = jnp.maximum(acc2 + b2_ref[:, 0:1], 0.0)

        for y in range(th):
            o_ref[0, :, pl.ds(y * w, w)] = y2[:, y * wp + 1:y * wp + 1 + w]

    out = pl.pallas_call(
        body,
        out_shape=jax.ShapeDtypeStruct((n, cout, h * w), x.dtype),
        grid_spec=pltpu.PrefetchScalarGridSpec(
            num_scalar_prefetch=0,
            grid=(n, nt),
            in_specs=[
                pl.BlockSpec(memory_space=pl.ANY),
                pl.BlockSpec((cout, 9 * cin), lambda b, t: (0, 0)),
                pl.BlockSpec((cout, 9 * cout), lambda b, t: (0, 0)),
                pl.BlockSpec((cout, 128), lambda b, t: (0, 0)),
                pl.BlockSpec((cout, 128), lambda b, t: (0, 0)),
                pl.BlockSpec((1, 1, r1 * wp), lambda b, t: (t, 0, 0)),
            ],
            out_specs=pl.BlockSpec((1, cout, th * w), lambda b, t: (b, 0, t)),
            scratch_shapes=[
                pltpu.VMEM((2, cin, frows, wp), jnp.bfloat16),
                pltpu.SemaphoreType.DMA((2,)),
            ]),
        compiler_params=pltpu.CompilerParams(
            dimension_semantics=("parallel", "arbitrary"),
            vmem_limit_bytes=100 * 1024 * 1024),
    )(xp, w1p, w2p, b1p, b2p, maskp)

    return out.reshape(n, cout, h, w)


# speedup vs baseline: 1.6409x; 1.2421x over previous
import jax
import jax.numpy as jnp
from jax.experimental import pallas as pl
from jax.experimental.pallas import tpu as pltpu


def _pick_row_tile(h, max_rows=32):
    for th in range(min(h, max_rows), 0, -1):
        if h % th == 0:
            return th
    return h


def _shifted(v, o, length):
    m = v.shape[1]
    if o < 0:
        return jnp.concatenate([v[:, :-o], v[:, :length + o]], axis=1)
    if o + length > m:
        return jnp.concatenate([v[:, o:], v[:, :o + length - m]], axis=1)
    return v[:, o:o + length]


def kernel(x, w1, b1, w2, b2):
    n, cin, h, w = x.shape
    cout = w1.shape[0]
    wp = w + 2
    th = _pick_row_tile(h)
    nt = h // th
    r1 = th + 2
    frows = -(-(th + 4) // 8) * 8
    xcols = frows * wp

    xp = jnp.pad(x, ((0, 0), (0, 0), (2, frows - th - 2), (1, 1)))
    xp = xp.astype(jnp.bfloat16)

    w1p = jnp.transpose(w1, (0, 2, 3, 1)).reshape(cout, 9 * cin)
    w1p = w1p.astype(jnp.bfloat16)
    w2p = jnp.transpose(w2, (0, 2, 3, 1)).reshape(cout, 9 * cout)
    w2p = w2p.astype(jnp.bfloat16)
    b1p = jnp.broadcast_to(b1.astype(jnp.float32)[:, None], (cout, 128))
    b2p = jnp.broadcast_to(b2.astype(jnp.float32)[:, None], (cout, 128))

    lane = jnp.arange(r1 * wp, dtype=jnp.int32)[None, None, :]
    col = lane % wp
    good = (col != 0) & (col != wp - 1)
    tt = jnp.arange(nt, dtype=jnp.int32)[:, None, None]
    good = good & ~((tt == 0) & (lane < wp))
    good = good & ~((tt == nt - 1) & (lane >= (r1 - 1) * wp))
    maskp = good.astype(jnp.bfloat16)

    def body(x_hbm, w1_ref, w2_ref, b1_ref, b2_ref, m_ref, o_ref, xbuf,
             in_sem):
        b = pl.program_id(0)
        t = pl.program_id(1)
        ntt = pl.num_programs(1)

        def fetch(tt, slot):
            pltpu.make_async_copy(x_hbm.at[b, :, pl.ds(tt * th, frows), :],
                                  xbuf.at[slot], in_sem.at[slot]).start()

        slot = jax.lax.rem(t, 2)

        @pl.when(t == 0)
        def _():
            fetch(t, slot)

        @pl.when(t + 1 < ntt)
        def _():
            fetch(t + 1, 1 - slot)

        pltpu.make_async_copy(x_hbm.at[b, :, pl.ds(0, frows), :],
                              xbuf.at[slot], in_sem.at[slot]).wait()
        xv = xbuf[slot].reshape(cin, xcols)

        patch1 = jnp.concatenate(
            [_shifted(xv, dy * wp + dx - 1, r1 * wp)
             for dy in range(3) for dx in range(3)], axis=0)
        acc1 = jnp.dot(w1_ref[...], patch1,
                       preferred_element_type=jnp.float32)

        mid = jnp.maximum(acc1 + b1_ref[:, 0:1], 0.0).astype(jnp.bfloat16)
        mid = mid * m_ref[0]

        patch2 = jnp.concatenate(
            [_shifted(mid, dy * wp + dx - 1, th * wp)
             for dy in range(3) for dx in range(3)], axis=0)
        acc2 = jnp.dot(w2_ref[...], patch2,
                       preferred_element_type=jnp.float32)
        y2 = jnp.maximum(acc2 + b2_ref[:, 0:1], 0.0)

        ext = jnp.concatenate(
            [y2[:, y * wp + 1:y * wp + 1 + w] for y in range(th)], axis=1)
        o_ref[0] = ext.reshape(cout, th, w)

    out = pl.pallas_call(
        body,
        out_shape=jax.ShapeDtypeStruct((n, cout, h, w), x.dtype),
        grid_spec=pltpu.PrefetchScalarGridSpec(
            num_scalar_prefetch=0,
            grid=(n, nt),
            in_specs=[
                pl.BlockSpec(memory_space=pl.ANY),
                pl.BlockSpec((cout, 9 * cin), lambda b, t: (0, 0)),
                pl.BlockSpec((cout, 9 * cout), lambda b, t: (0, 0)),
                pl.BlockSpec((cout, 128), lambda b, t: (0, 0)),
                pl.BlockSpec((cout, 128), lambda b, t: (0, 0)),
                pl.BlockSpec((1, 1, r1 * wp), lambda b, t: (t, 0, 0)),
            ],
            out_specs=pl.BlockSpec((1, cout, th, w),
                                   lambda b, t: (b, 0, t, 0)),
            scratch_shapes=[
                pltpu.VMEM((2, cin, frows, wp), jnp.bfloat16),
                pltpu.SemaphoreType.DMA((2,)),
            ]),
        compiler_params=pltpu.CompilerParams(
            dimension_semantics=("parallel", "arbitrary"),
            vmem_limit_bytes=100 * 1024 * 1024),
    )(xp, w1p, w2p, b1p, b2p, maskp)

    return out


# speedup vs baseline: 1.7229x; 1.0500x over previous
import jax
import jax.numpy as jnp
from jax.experimental import pallas as pl
from jax.experimental.pallas import tpu as pltpu


def _pick_row_tile(h, max_rows=32):
    for th in range(min(h, max_rows), 0, -1):
        if h % th == 0:
            return th
    return h


def _shifted(v, o, length):
    m = v.shape[1]
    if o < 0:
        return jnp.concatenate([v[:, :-o], v[:, :length + o]], axis=1)
    if o + length > m:
        return jnp.concatenate([v[:, o:], v[:, :o + length - m]], axis=1)
    return v[:, o:o + length]


def kernel(x, w1, b1, w2, b2):
    n, cin, h, w = x.shape
    cout = w1.shape[0]
    th = _pick_row_tile(h)
    nt = h // th
    r1 = th + 2
    l1 = r1 * w
    l2 = th * w
    fsize = -(-((th + 4) * w + 2) // 128) * 128
    flat_len = (nt - 1) * th * w + fsize

    xp = jnp.pad(x, ((0, 0), (0, 0), (2, 2), (0, 0))).astype(jnp.bfloat16)
    xp = xp.reshape(n, cin, (h + 4) * w)
    xp = jnp.pad(xp, ((0, 0), (0, 0), (0, flat_len - (h + 4) * w)))

    w1p = jnp.transpose(w1, (0, 2, 3, 1)).reshape(cout, 9 * cin)
    w1p = w1p.astype(jnp.bfloat16)
    w2p = jnp.transpose(w2, (0, 2, 3, 1)).reshape(cout, 9 * cout)
    w2p = w2p.astype(jnp.bfloat16)
    b1p = jnp.broadcast_to(b1.astype(jnp.float32)[:, None], (cout, 128))
    b2p = jnp.broadcast_to(b2.astype(jnp.float32)[:, None], (cout, 128))

    col1 = jnp.arange(l1, dtype=jnp.int32) % w
    ml1 = (col1 != 0).astype(jnp.bfloat16)[None]
    mr1 = (col1 != w - 1).astype(jnp.bfloat16)[None]
    col2 = jnp.arange(l2, dtype=jnp.int32) % w
    ml2 = (col2 != 0).astype(jnp.bfloat16)[None]
    mr2 = (col2 != w - 1).astype(jnp.bfloat16)[None]

    lane = jnp.arange(l1, dtype=jnp.int32)[None, None, :]
    tt = jnp.arange(nt, dtype=jnp.int32)[:, None, None]
    good = ~((tt == 0) & (lane < w))
    good = good & ~((tt == nt - 1) & (lane >= (r1 - 1) * w))
    maskp = good.astype(jnp.bfloat16)

    def body(x_hbm, w1_ref, w2_ref, b1_ref, b2_ref, m_ref,
             ml1_ref, mr1_ref, ml2_ref, mr2_ref, o_ref, xbuf, in_sem):
        b = pl.program_id(0)
        t = pl.program_id(1)
        ntt = pl.num_programs(1)

        def fetch(ti, slot):
            pltpu.make_async_copy(x_hbm.at[b, :, pl.ds(ti * th * w, fsize)],
                                  xbuf.at[slot], in_sem.at[slot]).start()

        slot = jax.lax.rem(t, 2)

        @pl.when(t == 0)
        def _():
            fetch(t, slot)

        @pl.when(t + 1 < ntt)
        def _():
            fetch(t + 1, 1 - slot)

        pltpu.make_async_copy(x_hbm.at[b, :, pl.ds(0, fsize)],
                              xbuf.at[slot], in_sem.at[slot]).wait()
        xv = xbuf[slot]

        def conv(src, wref, length, mleft, mright):
            pieces = []
            for dy in range(3):
                for dx in range(3):
                    p = _shifted(src, dy * w + dx - 1, length)
                    if dx == 0:
                        p = p * mleft
                    elif dx == 2:
                        p = p * mright
                    pieces.append(p)
            return jnp.dot(wref[...], jnp.concatenate(pieces, axis=0),
                           preferred_element_type=jnp.float32)

        acc1 = conv(xv, w1_ref, l1, ml1_ref[...], mr1_ref[...])
        mid = jnp.maximum(acc1 + b1_ref[:, 0:1], 0.0).astype(jnp.bfloat16)
        mid = mid * m_ref[0]

        acc2 = conv(mid, w2_ref, l2, ml2_ref[...], mr2_ref[...])
        y2 = jnp.maximum(acc2 + b2_ref[:, 0:1], 0.0)
        o_ref[0] = y2.reshape(cout, th, w)

    out = pl.pallas_call(
        body,
        out_shape=jax.ShapeDtypeStruct((n, cout, h, w), x.dtype),
        grid_spec=pltpu.PrefetchScalarGridSpec(
            num_scalar_prefetch=0,
            grid=(n, nt),
            in_specs=[
                pl.BlockSpec(memory_space=pl.ANY),
                pl.BlockSpec((cout, 9 * cin), lambda b, t: (0, 0)),
                pl.BlockSpec((cout, 9 * cout), lambda b, t: (0, 0)),
                pl.BlockSpec((cout, 128), lambda b, t: (0, 0)),
                pl.BlockSpec((cout, 128), lambda b, t: (0, 0)),
                pl.BlockSpec((1, 1, l1), lambda b, t: (t, 0, 0)),
                pl.BlockSpec((1, l1), lambda b, t: (0, 0)),
                pl.BlockSpec((1, l1), lambda b, t: (0, 0)),
                pl.BlockSpec((1, l2), lambda b, t: (0, 0)),
                pl.BlockSpec((1, l2), lambda b, t: (0, 0)),
            ],
            out_specs=pl.BlockSpec((1, cout, th, w),
                                   lambda b, t: (b, 0, t, 0)),
            scratch_shapes=[
                pltpu.VMEM((2, cin, fsize), jnp.bfloat16),
                pltpu.SemaphoreType.DMA((2,)),
            ]),
        compiler_params=pltpu.CompilerParams(
            dimension_semantics=("parallel", "arbitrary"),
            vmem_limit_bytes=100 * 1024 * 1024),
    )(xp, w1p, w2p, b1p, b2p, maskp, ml1, mr1, ml2, mr2)

    return out


# speedup vs baseline: 2.2209x; 1.2890x over previous
import jax
import jax.numpy as jnp
from jax.experimental import pallas as pl
from jax.experimental.pallas import tpu as pltpu


def _pick_row_tile(h, max_rows=32):
    for th in range(min(h, max_rows), 0, -1):
        if h % th == 0:
            return th
    return h


def _shifted(v, o, length):
    m = v.shape[1]
    if o < 0:
        z = jnp.zeros((v.shape[0], -o), v.dtype)
        return jnp.concatenate([z, v[:, :length + o]], axis=1)
    if o + length > m:
        z = jnp.zeros((v.shape[0], o + length - m), v.dtype)
        return jnp.concatenate([v[:, o:], z], axis=1)
    return v[:, o:o + length]


def kernel(x, w1, b1, w2, b2):
    n, cin, h, w = x.shape
    cout = w1.shape[0]
    th = _pick_row_tile(h)
    nt = h // th
    r1 = th + 2
    l1 = r1 * w
    l2 = th * w
    fsize = -(-((th + 4) * w + 2) // 128) * 128
    flat_len = (nt - 1) * th * w + fsize

    xp = jnp.pad(x, ((0, 0), (0, 0), (2, 2), (0, 0))).astype(jnp.bfloat16)
    xp = xp.reshape(n, cin, (h + 4) * w)
    xp = jnp.pad(xp, ((0, 0), (0, 0), (0, flat_len - (h + 4) * w)))

    w1p = jnp.transpose(w1, (0, 2, 3, 1)).reshape(cout, 9 * cin)
    w1p = w1p.astype(jnp.bfloat16)
    w2p = jnp.transpose(w2, (0, 2, 3, 1)).reshape(cout, 9 * cout)
    w2p = w2p.astype(jnp.bfloat16)
    b1p = jnp.broadcast_to(b1.astype(jnp.float32)[:, None], (cout, 128))
    b2p = jnp.broadcast_to(b2.astype(jnp.float32)[:, None], (cout, 128))

    s1 = jnp.arange(fsize, dtype=jnp.int32) % w
    ma1 = (s1 != w - 1).astype(jnp.bfloat16)[None]
    mb1 = (s1 != 0).astype(jnp.bfloat16)[None]
    s2 = jnp.arange(l1, dtype=jnp.int32) % w
    ma2 = (s2 != w - 1).astype(jnp.bfloat16)[None]
    mb2 = (s2 != 0).astype(jnp.bfloat16)[None]

    lane = jnp.arange(l1, dtype=jnp.int32)[None, None, :]
    tt = jnp.arange(nt, dtype=jnp.int32)[:, None, None]
    good = ~((tt == 0) & (lane < w))
    good = good & ~((tt == nt - 1) & (lane >= (r1 - 1) * w))
    maskp = good.astype(jnp.bfloat16)

    def body(x_hbm, w1_ref, w2_ref, b1_ref, b2_ref, m_ref,
             ma1_ref, mb1_ref, ma2_ref, mb2_ref, o_ref, xbuf, in_sem):
        b = pl.program_id(0)
        t = pl.program_id(1)
        ntt = pl.num_programs(1)

        def fetch(ti, slot):
            pltpu.make_async_copy(x_hbm.at[b, :, pl.ds(ti * th * w, fsize)],
                                  xbuf.at[slot], in_sem.at[slot]).start()

        slot = jax.lax.rem(t, 2)

        @pl.when(t == 0)
        def _():
            fetch(t, slot)

        @pl.when(t + 1 < ntt)
        def _():
            fetch(t + 1, 1 - slot)

        pltpu.make_async_copy(x_hbm.at[b, :, pl.ds(0, fsize)],
                              xbuf.at[slot], in_sem.at[slot]).wait()
        xv = xbuf[slot]

        def conv(src, wref, length, mleft, mright):
            srcs = (src * mleft, src, src * mright)
            pieces = [_shifted(srcs[dx], dy * w + dx - 1, length)
                      for dy in range(3) for dx in range(3)]
            return jnp.dot(wref[...], jnp.concatenate(pieces, axis=0),
                           preferred_element_type=jnp.float32)

        acc1 = conv(xv, w1_ref, l1, ma1_ref[...], mb1_ref[...])
        mid = jnp.maximum(acc1 + b1_ref[:, 0:1], 0.0).astype(jnp.bfloat16)
        mid = mid * m_ref[0]

        acc2 = conv(mid, w2_ref, l2, ma2_ref[...], mb2_ref[...])
        y2 = jnp.maximum(acc2 + b2_ref[:, 0:1], 0.0)
        o_ref[0] = y2.reshape(cout, th, w)

    out = pl.pallas_call(
        body,
        out_shape=jax.ShapeDtypeStruct((n, cout, h, w), x.dtype),
        grid_spec=pltpu.PrefetchScalarGridSpec(
            num_scalar_prefetch=0,
            grid=(n, nt),
            in_specs=[
                pl.BlockSpec(memory_space=pl.ANY),
                pl.BlockSpec((cout, 9 * cin), lambda b, t: (0, 0)),
                pl.BlockSpec((cout, 9 * cout), lambda b, t: (0, 0)),
                pl.BlockSpec((cout, 128), lambda b, t: (0, 0)),
                pl.BlockSpec((cout, 128), lambda b, t: (0, 0)),
                pl.BlockSpec((1, 1, l1), lambda b, t: (t, 0, 0)),
                pl.BlockSpec((1, fsize), lambda b, t: (0, 0)),
                pl.BlockSpec((1, fsize), lambda b, t: (0, 0)),
                pl.BlockSpec((1, l1), lambda b, t: (0, 0)),
                pl.BlockSpec((1, l1), lambda b, t: (0, 0)),
            ],
            out_specs=pl.BlockSpec((1, cout, th, w),
                                   lambda b, t: (b, 0, t, 0)),
            scratch_shapes=[
                pltpu.VMEM((2, cin, fsize), jnp.bfloat16),
                pltpu.SemaphoreType.DMA((2,)),
            ]),
        compiler_params=pltpu.CompilerParams(
            dimension_semantics=("parallel", "arbitrary"),
            vmem_limit_bytes=100 * 1024 * 1024),
    )(xp, w1p, w2p, b1p, b2p, maskp, ma1, mb1, ma2, mb2)

    return out


# speedup vs baseline: 2.2456x; 1.0111x over previous
import jax
import jax.numpy as jnp
from jax.experimental import pallas as pl
from jax.experimental.pallas import tpu as pltpu


def _pick_row_tile(h, max_rows=64):
    for th in range(min(h, max_rows), 0, -1):
        if h % th == 0:
            return th
    return h


def _shifted(v, o, length):
    m = v.shape[1]
    if o < 0:
        z = jnp.zeros((v.shape[0], -o), v.dtype)
        return jnp.concatenate([z, v[:, :length + o]], axis=1)
    if o + length > m:
        z = jnp.zeros((v.shape[0], o + length - m), v.dtype)
        return jnp.concatenate([v[:, o:], z], axis=1)
    return v[:, o:o + length]


def kernel(x, w1, b1, w2, b2):
    n, cin, h, w = x.shape
    cout = w1.shape[0]
    th = _pick_row_tile(h)
    nt = h // th
    r1 = th + 2
    l1 = r1 * w
    l2 = th * w
    fsize = -(-((th + 4) * w + 2) // 128) * 128
    flat_len = (nt - 1) * th * w + fsize

    xp = jnp.pad(x, ((0, 0), (0, 0), (2, 2), (0, 0))).astype(jnp.bfloat16)
    xp = xp.reshape(n, cin, (h + 4) * w)
    xp = jnp.pad(xp, ((0, 0), (0, 0), (0, flat_len - (h + 4) * w)))

    w1p = jnp.transpose(w1, (0, 2, 3, 1)).reshape(cout, 9 * cin)
    w1p = w1p.astype(jnp.bfloat16)
    w2p = jnp.transpose(w2, (0, 2, 3, 1)).reshape(cout, 9 * cout)
    w2p = w2p.astype(jnp.bfloat16)
    b1p = jnp.broadcast_to(b1.astype(jnp.float32)[:, None], (cout, 128))
    b2p = jnp.broadcast_to(b2.astype(jnp.float32)[:, None], (cout, 128))

    s1 = jnp.arange(fsize, dtype=jnp.int32) % w
    ma1 = (s1 != w - 1).astype(jnp.bfloat16)[None]
    mb1 = (s1 != 0).astype(jnp.bfloat16)[None]
    s2 = jnp.arange(l1, dtype=jnp.int32) % w
    ma2 = (s2 != w - 1).astype(jnp.bfloat16)[None]
    mb2 = (s2 != 0).astype(jnp.bfloat16)[None]

    lane = jnp.arange(l1, dtype=jnp.int32)[None, None, :]
    tt = jnp.arange(nt, dtype=jnp.int32)[:, None, None]
    good = ~((tt == 0) & (lane < w))
    good = good & ~((tt == nt - 1) & (lane >= (r1 - 1) * w))
    maskp = good.astype(jnp.bfloat16)

    def body(x_hbm, w1_ref, w2_ref, b1_ref, b2_ref, m_ref,
             ma1_ref, mb1_ref, ma2_ref, mb2_ref, o_ref, xbuf, in_sem):
        b = pl.program_id(0)
        t = pl.program_id(1)
        ntt = pl.num_programs(1)

        def fetch(ti, slot):
            pltpu.make_async_copy(x_hbm.at[b, :, pl.ds(ti * th * w, fsize)],
                                  xbuf.at[slot], in_sem.at[slot]).start()

        slot = jax.lax.rem(t, 2)

        @pl.when(t == 0)
        def _():
            fetch(t, slot)

        @pl.when(t + 1 < ntt)
        def _():
            fetch(t + 1, 1 - slot)

        pltpu.make_async_copy(x_hbm.at[b, :, pl.ds(0, fsize)],
                              xbuf.at[slot], in_sem.at[slot]).wait()
        xv = xbuf[slot]

        def conv(src, wref, length, mleft, mright):
            srcs = (src * mleft, src, src * mright)
            pieces = [_shifted(srcs[dx], dy * w + dx - 1, length)
                      for dy in range(3) for dx in range(3)]
            return jnp.dot(wref[...], jnp.concatenate(pieces, axis=0),
                           preferred_element_type=jnp.float32)

        acc1 = conv(xv, w1_ref, l1, ma1_ref[...], mb1_ref[...])
        mid = jnp.maximum(acc1 + b1_ref[:, 0:1], 0.0).astype(jnp.bfloat16)
        mid = mid * m_ref[0]

        acc2 = conv(mid, w2_ref, l2, ma2_ref[...], mb2_ref[...])
        y2 = jnp.maximum(acc2 + b2_ref[:, 0:1], 0.0)
        o_ref[0] = y2.reshape(cout, th, w)

    out = pl.pallas_call(
        body,
        out_shape=jax.ShapeDtypeStruct((n, cout, h, w), x.dtype),
        grid_spec=pltpu.PrefetchScalarGridSpec(
            num_scalar_prefetch=0,
            grid=(n, nt),
            in_specs=[
                pl.BlockSpec(memory_space=pl.ANY),
                pl.BlockSpec((cout, 9 * cin), lambda b, t: (0, 0)),
                pl.BlockSpec((cout, 9 * cout), lambda b, t: (0, 0)),
                pl.BlockSpec((cout, 128), lambda b, t: (0, 0)),
                pl.BlockSpec((cout, 128), lambda b, t: (0, 0)),
                pl.BlockSpec((1, 1, l1), lambda b, t: (t, 0, 0)),
                pl.BlockSpec((1, fsize), lambda b, t: (0, 0)),
                pl.BlockSpec((1, fsize), lambda b, t: (0, 0)),
                pl.BlockSpec((1, l1), lambda b, t: (0, 0)),
                pl.BlockSpec((1, l1), lambda b, t: (0, 0)),
            ],
            out_specs=pl.BlockSpec((1, cout, th, w),
                                   lambda b, t: (b, 0, t, 0)),
            scratch_shapes=[
                pltpu.VMEM((2, cin, fsize), jnp.bfloat16),
                pltpu.SemaphoreType.DMA((2,)),
            ]),
        compiler_params=pltpu.CompilerParams(
            dimension_semantics=("parallel", "arbitrary"),
            vmem_limit_bytes=100 * 1024 * 1024),
    )(xp, w1p, w2p, b1p, b2p, maskp, ma1, mb1, ma2, mb2)

    return out
